# Initial kernel scaffold; baseline (speedup 1.0000x reference)
import jax
import jax.numpy as jnp
from jax.experimental import pallas as pl


def kernel(nodes, params, indexes):
    raise NotImplementedError("write your pallas kernel here")


# speedup vs baseline: 1.3711x; 1.3711x over previous
import functools

import jax
import jax.numpy as jnp
from jax import lax
from jax.experimental import pallas as pl
from jax.experimental.pallas import tpu as pltpu
from jax.experimental.pallas import tpu_sc as plsc

N = 50000
E = 40000
D = 128
NUM_REL = 14
NB = 1000
GRID = N // NB
NBD = 1000
GRIDD = N // NBD
B_E = 128
N_TILES = 16
EPAD = 40960
NBATCH = EPAD // (N_TILES * B_E)
NPLANE = 6
NROUND = NPLANE // 2
Q = 9000
ZROWS = 568
SP_ROWS = N_TILES * ZROWS
ZCHUNK = 128


def _gn_rows(x, w, b):
    mu = jnp.mean(x, axis=1, keepdims=True)
    xc = x - mu
    var = jnp.mean(xc * xc, axis=1, keepdims=True)
    return xc * lax.rsqrt(var + 1e-5) * w + b


def _stage_a_body(nodes, iw1, ib1, iw2, ignw, ignb, sw1, sb1, sw2, sgnw,
                  sgnb, mw, mgnw, mgnb, wcat, feat_out, y_out):
    x = nodes[...]
    f = jnp.maximum(x[:, 0:1] * iw1[0:1, :] + x[:, 1:2] * iw1[1:2, :]
                    + ib1[...], 0.0)
    f = _gn_rows(jnp.dot(f, iw2[...], preferred_element_type=jnp.float32),
                 ignw[...], ignb[...])
    s = jnp.maximum(x[:, 2:3] * sw1[0:1, :] + x[:, 3:4] * sw1[1:2, :]
                    + sb1[...], 0.0)
    s = _gn_rows(jnp.dot(s, sw2[...], preferred_element_type=jnp.float32),
                 sgnw[...], sgnb[...])
    f = jnp.maximum(f + s, 0.0)
    m = jnp.dot(f, mw[0:D, :], preferred_element_type=jnp.float32)
    m = (m + x[:, 4:5] * mw[D:D + 1, :] + x[:, 5:6] * mw[D + 1:D + 2, :]
         + x[:, 6:7] * mw[D + 2:D + 3, :] + x[:, 7:8] * mw[D + 3:D + 4, :])
    f = jnp.maximum(_gn_rows(m, mgnw[...], mgnb[...]), 0.0)
    feat_out[...] = f
    y_big = jnp.dot(f, wcat[...], preferred_element_type=jnp.float32)
    for k in range(NUM_REL + 1):
        y_out[k] = y_big[:, k * D:(k + 1) * D]


def _stage_a(nodes, pv):
    full = lambda shp: pl.BlockSpec(shp, lambda i: tuple(0 for _ in shp))
    return pl.pallas_call(
        _stage_a_body,
        grid=(GRID,),
        in_specs=[
            pl.BlockSpec((NB, 8), lambda i: (i, 0)),
            full((2, D)), full((1, D)), full((D, D)), full((1, D)),
            full((1, D)),
            full((2, D)), full((1, D)), full((D, D)), full((1, D)),
            full((1, D)),
            full((D + 4, D)), full((1, D)), full((1, D)),
            full((D, (NUM_REL + 1) * D)),
        ],
        out_specs=[
            pl.BlockSpec((NB, D), lambda i: (i, 0)),
            pl.BlockSpec((NUM_REL + 1, NB, D), lambda i: (0, i, 0)),
        ],
        out_shape=[
            jax.ShapeDtypeStruct((N, D), jnp.float32),
            jax.ShapeDtypeStruct((NUM_REL + 1, N, D), jnp.float32),
        ],
    )(nodes, *pv)


def _stage_d_feat(t, y, res, nw, nb_, c2w, c2gw, c2gb):
    temp = t[0] + y[0]
    x1 = jnp.maximum(_gn_rows(temp, nw[...], nb_[...]), 0.0)
    z = _gn_rows(jnp.dot(x1, c2w[...], preferred_element_type=jnp.float32),
                 c2gw[...], c2gb[...])
    return jnp.maximum(z + res[...], 0.0)


def _stage_d_body(t, y, res, nw, nb_, c2w, c2gw, c2gb, wcat, feat_out,
                  y_out):
    f = _stage_d_feat(t, y, res, nw, nb_, c2w, c2gw, c2gb)
    feat_out[...] = f
    y_big = jnp.dot(f, wcat[...], preferred_element_type=jnp.float32)
    for k in range(NUM_REL + 1):
        y_out[k] = y_big[:, k * D:(k + 1) * D]


def _stage_d_body_last(t, y, res, nw, nb_, c2w, c2gw, c2gb, feat_out):
    feat_out[...] = _stage_d_feat(t, y, res, nw, nb_, c2w, c2gw, c2gb)


_BPP = Q // NBD


def _stage_d(temp4, y, res, nw, nb_, c2w, c2gw, c2gb, wcat):
    last = wcat is None
    full = lambda shp: pl.BlockSpec(shp, lambda i: tuple(0 for _ in shp))
    body = _stage_d_body_last if last else _stage_d_body
    in_specs = [
        pl.BlockSpec((1, NBD, D), lambda i: (i // _BPP, i % _BPP, 0)),
        pl.BlockSpec((1, NBD, D), lambda i: (0, i, 0)),
        pl.BlockSpec((NBD, D), lambda i: (i, 0)),
        full((1, D)), full((1, D)), full((D, D)), full((1, D)),
        full((1, D)),
    ]
    out_specs = [pl.BlockSpec((NBD, D), lambda i: (i, 0))]
    out_shape = [jax.ShapeDtypeStruct((N, D), jnp.float32)]
    args = [temp4, y, res, nw, nb_, c2w, c2gw, c2gb]
    if not last:
        in_specs.append(full((D, (NUM_REL + 1) * D)))
        out_specs.append(pl.BlockSpec((NUM_REL + 1, NBD, D),
                                      lambda i: (0, i, 0)))
        out_shape.append(
            jax.ShapeDtypeStruct((NUM_REL + 1, N, D), jnp.float32))
        args.append(wcat)
    res_ = pl.pallas_call(
        body, grid=(GRIDD,), in_specs=in_specs, out_specs=out_specs,
        out_shape=out_shape,
    )(*args)
    return res_ if not last else (res_[0], None)


_sc_mesh = plsc.VectorSubcoreMesh(core_axis_name="c", subcore_axis_name="s",
                                  num_cores=2)


@functools.partial(
    pl.kernel,
    mesh=_sc_mesh,
    out_type=jax.ShapeDtypeStruct((NPLANE, SP_ROWS, D), jnp.float32),
    scratch_types=[
        pltpu.VMEM((NBATCH, B_E), jnp.int32),
        pltpu.VMEM((NBATCH, B_E), jnp.int32),
        pltpu.VMEM((B_E, D), jnp.float32),
        pltpu.VMEM((ZCHUNK, D), jnp.float32),
        pltpu.VMEM_SHARED((SP_ROWS, D), jnp.float32),
    ],
)
def _sc_scatter(y_hbm, src_hbm, dst_hbm, out_hbm, src_v, dst_v, rows_v,
                zero_v, acc_sh):
    cid = lax.axis_index("c")
    sid = lax.axis_index("s")

    zv = jnp.zeros((16,), jnp.float32)

    def zfill(k, carry):
        for c in range(D // 16):
            zero_v[k, 16 * c:16 * (c + 1)] = zv
        return carry

    lax.fori_loop(0, ZCHUNK, zfill, 0)

    zbase = sid * ZROWS

    for r in range(NROUND):
        q = cid + 2 * r

        def zcopy(k, carry):
            pltpu.sync_copy(zero_v, acc_sh.at[pl.ds(zbase + k * ZCHUNK,
                                                    ZCHUNK)])
            return carry

        lax.fori_loop(0, ZROWS // ZCHUNK, zcopy, 0)
        rem = ZROWS % ZCHUNK
        pltpu.sync_copy(zero_v.at[pl.ds(0, rem)],
                        acc_sh.at[pl.ds(zbase + (ZROWS // ZCHUNK) * ZCHUNK,
                                        rem)])
        plsc.subcore_barrier()

        def rel_body(j, carry):
            pltpu.sync_copy(src_hbm.at[j, sid], src_v)
            pltpu.sync_copy(dst_hbm.at[q, j, sid], dst_v)

            def batch_body(b, c2):
                pltpu.sync_copy(y_hbm.at[src_v.at[b]], rows_v)
                pltpu.sync_copy(rows_v, acc_sh.at[dst_v.at[b]], add=True)
                return c2

            lax.fori_loop(0, NBATCH, batch_body, 0)
            return carry

        lax.fori_loop(0, NUM_REL, rel_body, 0)
        plsc.subcore_barrier()

        pltpu.sync_copy(acc_sh.at[pl.ds(zbase, ZROWS)],
                        out_hbm.at[q, pl.ds(zbase, ZROWS)])
        if r < NROUND - 1:
            plsc.subcore_barrier()


def _prep_indices(indexes):
    idx = indexes.astype(jnp.int32)
    dst = idx[:, 0::2].T
    src = idx[:, 1::2].T
    pad = EPAD - E
    ar = jnp.arange(pad, dtype=jnp.int32)
    pad_src = (ar * 977) % N
    pad_dst = N + (ar % 960)
    srcp = jnp.concatenate(
        [src, jnp.broadcast_to(pad_src, (NUM_REL, pad))], axis=1)
    dstp = jnp.concatenate(
        [dst, jnp.broadcast_to(pad_dst, (NUM_REL, pad))], axis=1)
    jrel = jnp.arange(NUM_REL, dtype=jnp.int32)[:, None]
    src_flat = (jrel + 1) * N + srcp
    src_flat = src_flat.reshape(NUM_REL, N_TILES, NBATCH, B_E)
    qs = jnp.arange(NPLANE, dtype=jnp.int32)[:, None, None] * Q
    loc = dstp[None] - qs
    e_sp = jnp.arange(EPAD, dtype=jnp.int32)[None, None, :] % (SP_ROWS - Q)
    dstq = jnp.where((loc >= 0) & (loc < Q), loc, Q + e_sp)
    dstq = dstq.reshape(NPLANE, NUM_REL, N_TILES, NBATCH, B_E)
    return src_flat, dstq


def kernel(nodes, params, indexes):
    p = params
    src_flat, dst_flat = _prep_indices(indexes)
    r1 = lambda a: a.reshape(1, D)
    wcats = [
        jnp.transpose(
            jnp.concatenate([p['ctr_w'][i:i + 1], p['rel_w'][i]], axis=0),
            (1, 0, 2)).reshape(D, (NUM_REL + 1) * D)
        for i in range(4)
    ]
    pv = [
        p['input_w1'], r1(p['input_b1']), p['input_w2'],
        r1(p['input_gn_w']), r1(p['input_gn_b']),
        p['seg_w1'], r1(p['seg_b1']), p['seg_w2'],
        r1(p['seg_gn_w']), r1(p['seg_gn_b']),
        p['meta_w'], r1(p['meta_gn_w']), r1(p['meta_gn_b']),
        wcats[0],
    ]
    feat, y = _stage_a(nodes, pv)
    for i in range(4):
        y_flat = y.reshape((NUM_REL + 1) * N, D)
        temp4 = _sc_scatter(y_flat, src_flat, dst_flat)
        wcat_next = wcats[i + 1] if i < 3 else None
        feat, y = _stage_d(
            temp4, y, feat,
            r1(p['norm_w'][i]), r1(p['norm_b'][i]),
            p['ctr2_w'][i],
            r1(p['ctr2_gn_w'][i]), r1(p['ctr2_gn_b'][i]),
            wcat_next)
    return (feat, nodes[:, :2])


# speedup vs baseline: 1.7726x; 1.2929x over previous
import functools

import jax
import jax.numpy as jnp
from jax import lax
from jax.experimental import pallas as pl
from jax.experimental.pallas import tpu as pltpu
from jax.experimental.pallas import tpu_sc as plsc

N = 50000
E = 40000
D = 128
NUM_REL = 14
NB = 1000
GRID = N // NB
NBD = 1000
GRIDD = N // NBD
B_E = 128
N_TILES = 16
EPAD = 40960
NBATCH = EPAD // (N_TILES * B_E)
NPLANE = 6
NROUND = NPLANE // 2
Q = 9000
ZROWS = 568
SP_ROWS = N_TILES * ZROWS
ZCHUNK = 128


def _gn_rows(x, w, b):
    mu = jnp.mean(x, axis=1, keepdims=True)
    xc = x - mu
    var = jnp.mean(xc * xc, axis=1, keepdims=True)
    return xc * lax.rsqrt(var + 1e-5) * w + b


def _stage_a_body(nodes, iw1, ib1, iw2, ignw, ignb, sw1, sb1, sw2, sgnw,
                  sgnb, mw, mgnw, mgnb, wcat, feat_out, y_out):
    x = nodes[...]
    f = jnp.maximum(x[:, 0:1] * iw1[0:1, :] + x[:, 1:2] * iw1[1:2, :]
                    + ib1[...], 0.0)
    f = _gn_rows(jnp.dot(f, iw2[...], preferred_element_type=jnp.float32),
                 ignw[...], ignb[...])
    s = jnp.maximum(x[:, 2:3] * sw1[0:1, :] + x[:, 3:4] * sw1[1:2, :]
                    + sb1[...], 0.0)
    s = _gn_rows(jnp.dot(s, sw2[...], preferred_element_type=jnp.float32),
                 sgnw[...], sgnb[...])
    f = jnp.maximum(f + s, 0.0)
    m = jnp.dot(f, mw[0:D, :], preferred_element_type=jnp.float32)
    m = (m + x[:, 4:5] * mw[D:D + 1, :] + x[:, 5:6] * mw[D + 1:D + 2, :]
         + x[:, 6:7] * mw[D + 2:D + 3, :] + x[:, 7:8] * mw[D + 3:D + 4, :])
    f = jnp.maximum(_gn_rows(m, mgnw[...], mgnb[...]), 0.0)
    feat_out[...] = f
    y_big = jnp.dot(f, wcat[...], preferred_element_type=jnp.float32)
    for k in range(NUM_REL + 1):
        y_out[k] = y_big[:, k * D:(k + 1) * D]


def _stage_a(nodes, pv):
    full = lambda shp: pl.BlockSpec(shp, lambda i: tuple(0 for _ in shp))
    return pl.pallas_call(
        _stage_a_body,
        grid=(GRID,),
        in_specs=[
            pl.BlockSpec((NB, 8), lambda i: (i, 0)),
            full((2, D)), full((1, D)), full((D, D)), full((1, D)),
            full((1, D)),
            full((2, D)), full((1, D)), full((D, D)), full((1, D)),
            full((1, D)),
            full((D + 4, D)), full((1, D)), full((1, D)),
            full((D, (NUM_REL + 1) * D)),
        ],
        out_specs=[
            pl.BlockSpec((NB, D), lambda i: (i, 0)),
            pl.BlockSpec((NUM_REL + 1, NB, D), lambda i: (0, i, 0)),
        ],
        out_shape=[
            jax.ShapeDtypeStruct((N, D), jnp.float32),
            jax.ShapeDtypeStruct((NUM_REL + 1, N, D), jnp.float32),
        ],
    )(nodes, *pv)


def _stage_d_feat(t, y, res, nw, nb_, c2w, c2gw, c2gb):
    temp = t[0] + y[0]
    x1 = jnp.maximum(_gn_rows(temp, nw[...], nb_[...]), 0.0)
    z = _gn_rows(jnp.dot(x1, c2w[...], preferred_element_type=jnp.float32),
                 c2gw[...], c2gb[...])
    return jnp.maximum(z + res[...], 0.0)


def _stage_d_body(t, y, res, nw, nb_, c2w, c2gw, c2gb, wcat, feat_out,
                  y_out):
    f = _stage_d_feat(t, y, res, nw, nb_, c2w, c2gw, c2gb)
    feat_out[...] = f
    y_big = jnp.dot(f, wcat[...], preferred_element_type=jnp.float32)
    for k in range(NUM_REL + 1):
        y_out[k] = y_big[:, k * D:(k + 1) * D]


def _stage_d_body_last(t, y, res, nw, nb_, c2w, c2gw, c2gb, feat_out):
    feat_out[...] = _stage_d_feat(t, y, res, nw, nb_, c2w, c2gw, c2gb)


_BPP = Q // NBD


def _stage_d(temp4, y, res, nw, nb_, c2w, c2gw, c2gb, wcat):
    last = wcat is None
    full = lambda shp: pl.BlockSpec(shp, lambda i: tuple(0 for _ in shp))
    body = _stage_d_body_last if last else _stage_d_body
    in_specs = [
        pl.BlockSpec((1, NBD, D), lambda i: (i // _BPP, i % _BPP, 0)),
        pl.BlockSpec((1, NBD, D), lambda i: (0, i, 0)),
        pl.BlockSpec((NBD, D), lambda i: (i, 0)),
        full((1, D)), full((1, D)), full((D, D)), full((1, D)),
        full((1, D)),
    ]
    out_specs = [pl.BlockSpec((NBD, D), lambda i: (i, 0))]
    out_shape = [jax.ShapeDtypeStruct((N, D), jnp.float32)]
    args = [temp4, y, res, nw, nb_, c2w, c2gw, c2gb]
    if not last:
        in_specs.append(full((D, (NUM_REL + 1) * D)))
        out_specs.append(pl.BlockSpec((NUM_REL + 1, NBD, D),
                                      lambda i: (0, i, 0)))
        out_shape.append(
            jax.ShapeDtypeStruct((NUM_REL + 1, N, D), jnp.float32))
        args.append(wcat)
    res_ = pl.pallas_call(
        body, grid=(GRIDD,), in_specs=in_specs, out_specs=out_specs,
        out_shape=out_shape,
    )(*args)
    return res_ if not last else (res_[0], None)


_sc_mesh = plsc.VectorSubcoreMesh(core_axis_name="c", subcore_axis_name="s",
                                  num_cores=2)


@functools.partial(
    pl.kernel,
    mesh=_sc_mesh,
    out_type=jax.ShapeDtypeStruct((NPLANE, SP_ROWS, D), jnp.float32),
    scratch_types=[
        pltpu.VMEM((NBATCH, B_E), jnp.int32),
        pltpu.VMEM((NBATCH, B_E), jnp.int32),
        pltpu.VMEM((B_E, D), jnp.float32),
        pltpu.VMEM((B_E, D), jnp.float32),
        pltpu.VMEM((ZCHUNK, D), jnp.float32),
        pltpu.VMEM_SHARED((SP_ROWS, D), jnp.float32),
        pltpu.SemaphoreType.DMA,
        pltpu.SemaphoreType.DMA,
    ],
)
def _sc_scatter(y_hbm, src_hbm, dst_hbm, out_hbm, src_v, dst_v, rows0_v,
                rows1_v, zero_v, acc_sh, sem0, sem1):
    cid = lax.axis_index("c")
    sid = lax.axis_index("s")

    zv = jnp.zeros((16,), jnp.float32)

    def zfill(k, carry):
        for c in range(D // 16):
            zero_v[k, 16 * c:16 * (c + 1)] = zv
        return carry

    lax.fori_loop(0, ZCHUNK, zfill, 0)

    zbase = sid * ZROWS

    for r in range(NROUND):
        q = cid + 2 * r

        def zcopy(k, carry):
            pltpu.sync_copy(zero_v, acc_sh.at[pl.ds(zbase + k * ZCHUNK,
                                                    ZCHUNK)])
            return carry

        lax.fori_loop(0, ZROWS // ZCHUNK, zcopy, 0)
        rem = ZROWS % ZCHUNK
        pltpu.sync_copy(zero_v.at[pl.ds(0, rem)],
                        acc_sh.at[pl.ds(zbase + (ZROWS // ZCHUNK) * ZCHUNK,
                                        rem)])
        plsc.subcore_barrier()

        def rel_body(j, carry):
            pltpu.sync_copy(src_hbm.at[j, sid], src_v)
            pltpu.sync_copy(dst_hbm.at[q, j, sid], dst_v)
            pltpu.async_copy(y_hbm.at[src_v.at[0]], rows0_v, sem0)

            def batch2_body(h, c2):
                b0 = 2 * h
                b1 = 2 * h + 1
                pltpu.make_async_copy(y_hbm.at[src_v.at[b0]], rows0_v,
                                      sem0).wait()
                pltpu.async_copy(y_hbm.at[src_v.at[b1]], rows1_v, sem1)
                pltpu.sync_copy(rows0_v, acc_sh.at[dst_v.at[b0]], add=True)
                pltpu.make_async_copy(y_hbm.at[src_v.at[b1]], rows1_v,
                                      sem1).wait()
                bn = jnp.minimum(b0 + 2, NBATCH - 1)
                pltpu.async_copy(y_hbm.at[src_v.at[bn]], rows0_v, sem0)
                pltpu.sync_copy(rows1_v, acc_sh.at[dst_v.at[b1]], add=True)
                return c2

            lax.fori_loop(0, NBATCH // 2, batch2_body, 0)
            pltpu.make_async_copy(y_hbm.at[src_v.at[0]], rows0_v,
                                  sem0).wait()
            return carry

        lax.fori_loop(0, NUM_REL, rel_body, 0)
        plsc.subcore_barrier()

        pltpu.sync_copy(acc_sh.at[pl.ds(zbase, ZROWS)],
                        out_hbm.at[q, pl.ds(zbase, ZROWS)])
        if r < NROUND - 1:
            plsc.subcore_barrier()


def _prep_indices(indexes):
    idx = indexes.astype(jnp.int32)
    dst = idx[:, 0::2].T
    src = idx[:, 1::2].T
    pad = EPAD - E
    ar = jnp.arange(pad, dtype=jnp.int32)
    pad_src = (ar * 977) % N
    pad_dst = N + (ar % 960)
    srcp = jnp.concatenate(
        [src, jnp.broadcast_to(pad_src, (NUM_REL, pad))], axis=1)
    dstp = jnp.concatenate(
        [dst, jnp.broadcast_to(pad_dst, (NUM_REL, pad))], axis=1)
    jrel = jnp.arange(NUM_REL, dtype=jnp.int32)[:, None]
    src_flat = (jrel + 1) * N + srcp
    src_flat = src_flat.reshape(NUM_REL, N_TILES, NBATCH, B_E)
    qs = jnp.arange(NPLANE, dtype=jnp.int32)[:, None, None] * Q
    loc = dstp[None] - qs
    e_sp = jnp.arange(EPAD, dtype=jnp.int32)[None, None, :] % (SP_ROWS - Q)
    dstq = jnp.where((loc >= 0) & (loc < Q), loc, Q + e_sp)
    dstq = dstq.reshape(NPLANE, NUM_REL, N_TILES, NBATCH, B_E)
    return src_flat, dstq


def kernel(nodes, params, indexes):
    p = params
    src_flat, dst_flat = _prep_indices(indexes)
    r1 = lambda a: a.reshape(1, D)
    wcats = [
        jnp.transpose(
            jnp.concatenate([p['ctr_w'][i:i + 1], p['rel_w'][i]], axis=0),
            (1, 0, 2)).reshape(D, (NUM_REL + 1) * D)
        for i in range(4)
    ]
    pv = [
        p['input_w1'], r1(p['input_b1']), p['input_w2'],
        r1(p['input_gn_w']), r1(p['input_gn_b']),
        p['seg_w1'], r1(p['seg_b1']), p['seg_w2'],
        r1(p['seg_gn_w']), r1(p['seg_gn_b']),
        p['meta_w'], r1(p['meta_gn_w']), r1(p['meta_gn_b']),
        wcats[0],
    ]
    feat, y = _stage_a(nodes, pv)
    for i in range(4):
        y_flat = y.reshape((NUM_REL + 1) * N, D)
        temp4 = _sc_scatter(y_flat, src_flat, dst_flat)
        wcat_next = wcats[i + 1] if i < 3 else None
        feat, y = _stage_d(
            temp4, y, feat,
            r1(p['norm_w'][i]), r1(p['norm_b'][i]),
            p['ctr2_w'][i],
            r1(p['ctr2_gn_w'][i]), r1(p['ctr2_gn_b'][i]),
            wcat_next)
    return (feat, nodes[:, :2])
